# baseline (speedup 1.0000x reference)
import functools

import jax
import jax.numpy as jnp
from jax import lax
from jax.experimental import pallas as pl
from jax.experimental.pallas import tpu as pltpu
from jax.experimental.pallas import tpu_sc as plsc

D_MODEL = 1024
N_KINDS = 2
LANES = 16
D_VECS = D_MODEL // LANES
CHUNK = 16
NBUF = 6
PREF = 3


def _make_sc_kernel(num_tokens):
    info = plsc.get_sparse_core_info()
    nc, ns = info.num_cores, info.num_subcores
    nw = nc * ns
    tok_per_w = num_tokens // nw
    n_chunks = tok_per_w // CHUNK
    n_groups = n_chunks // NBUF
    n_tail = n_chunks - n_groups * NBUF

    mesh = plsc.VectorSubcoreMesh(core_axis_name="c", subcore_axis_name="s")

    scratch = [
        pltpu.VMEM((tok_per_w,), jnp.int32),
        pltpu.VMEM((tok_per_w,), jnp.int32),
    ]
    scratch += [pltpu.VMEM((CHUNK, D_MODEL), jnp.float32) for _ in range(NBUF)]
    scratch += [
        pltpu.VMEM((N_KINDS, D_MODEL), jnp.float32),
        pltpu.VMEM((1, D_MODEL), jnp.float32),
    ]
    scratch += [pltpu.SemaphoreType.DMA for _ in range(2 * NBUF)]

    @functools.partial(
        pl.kernel,
        mesh=mesh,
        out_type=jax.ShapeDtypeStruct((num_tokens, D_MODEL), jnp.float32),
        scratch_types=scratch,
    )
    def sc_kernel(t_hbm, kind_hbm, time_hbm, view_hbm, kind_emb_hbm, out_hbm,
                  t_all, k_all, *rest):
        bufs = rest[:NBUF]
        comb = rest[NBUF]
        viewv = rest[NBUF + 1]
        gsem = rest[NBUF + 2:2 * NBUF + 2]
        wsem = rest[2 * NBUF + 2:]
        wid = lax.axis_index("s") * nc + lax.axis_index("c")
        w_base = wid * tok_per_w

        pltpu.sync_copy(t_hbm.at[pl.ds(w_base, tok_per_w)], t_all)
        pltpu.sync_copy(kind_hbm.at[pl.ds(w_base, tok_per_w)], k_all)

        pltpu.sync_copy(kind_emb_hbm, comb)
        pltpu.sync_copy(view_hbm, viewv)
        for d in range(D_VECS):
            vv = viewv[0, pl.ds(d * LANES, LANES)]
            for k in range(N_KINDS):
                plsc.addupdate(comb.at[k, pl.ds(d * LANES, LANES)], vv)

        def gather(c, b):
            tv = t_all[pl.ds(c * CHUNK, CHUNK)]
            return pltpu.async_copy(time_hbm.at[tv], bufs[b], gsem[b])

        for c0 in range(PREF):
            gather(c0, c0)

        def emit_turn(c, u, is_static):
            bg = (u + PREF) % NBUF

            def drain():
                pltpu.make_async_copy(bufs[bg], out_hbm.at[pl.ds(0, CHUNK)],
                                      wsem[bg]).wait()

            def prefetch():
                gather(c + PREF, bg)

            if is_static:
                if c + PREF >= NBUF:
                    drain()
                if c + PREF < n_chunks:
                    prefetch()
            else:
                pl.when(c + PREF >= NBUF)(drain)
                pl.when(c + PREF < n_chunks)(prefetch)

            pltpu.make_async_copy(time_hbm.at[t_all[pl.ds(0, CHUNK)]],
                                  bufs[u], gsem[u]).wait()
            kvec = k_all[pl.ds(c * CHUNK, CHUNK)]
            kjs = [kvec[l] for l in range(LANES)]

            def d_body(d, c2, _kjs=kjs, _u=u):
                sl = pl.ds(d * LANES, LANES)
                vals = [comb[_kjs[l], sl] for l in range(LANES)]
                for l in range(LANES):
                    plsc.addupdate(bufs[_u].at[l, sl], vals[l])
                return c2

            lax.fori_loop(0, D_VECS, d_body, 0, unroll=2)
            pltpu.async_copy(bufs[u],
                             out_hbm.at[pl.ds(w_base + c * CHUNK, CHUNK)],
                             wsem[u])

        def group_body(g, carry):
            for u in range(NBUF):
                emit_turn(g * NBUF + u, u, False)
            return carry

        lax.fori_loop(0, n_groups, group_body, 0, unroll=False)
        for i in range(n_tail):
            emit_turn(n_groups * NBUF + i, i, True)

        for w in range(n_chunks - (NBUF - PREF), n_chunks):
            pltpu.make_async_copy(bufs[w % NBUF], out_hbm.at[pl.ds(0, CHUNK)],
                                  wsem[w % NBUF]).wait()

    return sc_kernel


def kernel(t, kind_id, view_id, time_emb, view_emb, kind_emb):
    b, s = t.shape
    n = b * s
    t_flat = t.reshape(n).astype(jnp.int32)
    kind_flat = kind_id.reshape(n).astype(jnp.int32)
    del view_id
    out = _make_sc_kernel(n)(t_flat, kind_flat, time_emb, view_emb, kind_emb)
    return out.reshape(b, s, D_MODEL)


# speedup vs baseline: 1.0381x; 1.0381x over previous
import functools

import jax
import jax.numpy as jnp
from jax import lax
from jax.experimental import pallas as pl
from jax.experimental.pallas import tpu as pltpu
from jax.experimental.pallas import tpu_sc as plsc

D_MODEL = 1024
N_KINDS = 2
LANES = 16
D_VECS = D_MODEL // LANES
CHUNK = 16
NBUF = 4
PREF = 2


def _make_sc_kernel(num_tokens):
    info = plsc.get_sparse_core_info()
    nc, ns = info.num_cores, info.num_subcores
    nw = nc * ns
    tok_per_w = num_tokens // nw
    n_chunks = tok_per_w // CHUNK
    n_groups = n_chunks // NBUF
    n_tail = n_chunks - n_groups * NBUF

    mesh = plsc.VectorSubcoreMesh(core_axis_name="c", subcore_axis_name="s")

    scratch = [
        pltpu.VMEM((tok_per_w,), jnp.int32),
        pltpu.VMEM((tok_per_w,), jnp.int32),
    ]
    scratch += [pltpu.VMEM((CHUNK, D_MODEL), jnp.float32) for _ in range(NBUF)]
    scratch += [
        pltpu.VMEM((N_KINDS, D_MODEL), jnp.float32),
        pltpu.VMEM((1, D_MODEL), jnp.float32),
    ]
    scratch += [pltpu.SemaphoreType.DMA for _ in range(2 * NBUF)]

    @functools.partial(
        pl.kernel,
        mesh=mesh,
        out_type=jax.ShapeDtypeStruct((num_tokens, D_MODEL), jnp.float32),
        scratch_types=scratch,
    )
    def sc_kernel(t_hbm, kind_hbm, time_hbm, view_hbm, kind_emb_hbm, out_hbm,
                  t_all, k_all, *rest):
        bufs = rest[:NBUF]
        comb = rest[NBUF]
        viewv = rest[NBUF + 1]
        gsem = rest[NBUF + 2:2 * NBUF + 2]
        wsem = rest[2 * NBUF + 2:]
        wid = lax.axis_index("s") * nc + lax.axis_index("c")
        w_base = wid * tok_per_w

        pltpu.sync_copy(t_hbm.at[pl.ds(w_base, tok_per_w)], t_all)

        def gather(c, b):
            tv = t_all[pl.ds(c * CHUNK, CHUNK)]
            return pltpu.async_copy(time_hbm.at[tv], bufs[b], gsem[b])

        for c0 in range(PREF):
            gather(c0, c0)

        pltpu.sync_copy(kind_hbm.at[pl.ds(w_base, tok_per_w)], k_all)
        pltpu.sync_copy(kind_emb_hbm, comb)
        pltpu.sync_copy(view_hbm, viewv)
        for d in range(D_VECS):
            vv = viewv[0, pl.ds(d * LANES, LANES)]
            for k in range(N_KINDS):
                plsc.addupdate(comb.at[k, pl.ds(d * LANES, LANES)], vv)

        def emit_turn(c, u, is_static):
            bg = (u + PREF) % NBUF

            def drain():
                pltpu.make_async_copy(bufs[bg], out_hbm.at[pl.ds(0, CHUNK)],
                                      wsem[bg]).wait()

            def prefetch():
                gather(c + PREF, bg)

            if is_static:
                if c + PREF >= NBUF:
                    drain()
                if c + PREF < n_chunks:
                    prefetch()
            else:
                pl.when(c + PREF >= NBUF)(drain)
                pl.when(c + PREF < n_chunks)(prefetch)

            pltpu.make_async_copy(time_hbm.at[t_all[pl.ds(0, CHUNK)]],
                                  bufs[u], gsem[u]).wait()
            kvec = k_all[pl.ds(c * CHUNK, CHUNK)]
            kjs = [kvec[l] for l in range(LANES)]

            def d_body(d, c2, _kjs=kjs, _u=u):
                sl = pl.ds(d * LANES, LANES)
                vals = [comb[_kjs[l], sl] for l in range(LANES)]
                for l in range(LANES):
                    plsc.addupdate(bufs[_u].at[l, sl], vals[l])
                return c2

            lax.fori_loop(0, D_VECS, d_body, 0, unroll=2)
            pltpu.async_copy(bufs[u],
                             out_hbm.at[pl.ds(w_base + c * CHUNK, CHUNK)],
                             wsem[u])

        def group_body(g, carry):
            for u in range(NBUF):
                emit_turn(g * NBUF + u, u, False)
            return carry

        lax.fori_loop(0, n_groups, group_body, 0, unroll=False)
        for i in range(n_tail):
            emit_turn(n_groups * NBUF + i, i, True)

        for w in range(n_chunks - (NBUF - PREF), n_chunks):
            pltpu.make_async_copy(bufs[w % NBUF], out_hbm.at[pl.ds(0, CHUNK)],
                                  wsem[w % NBUF]).wait()

    return sc_kernel


def kernel(t, kind_id, view_id, time_emb, view_emb, kind_emb):
    b, s = t.shape
    n = b * s
    t_flat = t.reshape(n).astype(jnp.int32)
    kind_flat = kind_id.reshape(n).astype(jnp.int32)
    del view_id
    out = _make_sc_kernel(n)(t_flat, kind_flat, time_emb, view_emb, kind_emb)
    return out.reshape(b, s, D_MODEL)


# speedup vs baseline: 1.0404x; 1.0023x over previous
import functools

import jax
import jax.numpy as jnp
from jax import lax
from jax.experimental import pallas as pl
from jax.experimental.pallas import tpu as pltpu
from jax.experimental.pallas import tpu_sc as plsc

D_MODEL = 1024
N_KINDS = 2
LANES = 16
D_VECS = D_MODEL // LANES
CHUNK = 16
NBUF = 4
PREF = 2


def _make_sc_kernel(num_tokens):
    info = plsc.get_sparse_core_info()
    nc, ns = info.num_cores, info.num_subcores
    nw = nc * ns
    tok_per_w = num_tokens // nw
    n_chunks = tok_per_w // CHUNK
    n_groups = n_chunks // NBUF
    n_tail = n_chunks - n_groups * NBUF

    mesh = plsc.VectorSubcoreMesh(core_axis_name="c", subcore_axis_name="s")

    scratch = [
        pltpu.VMEM((tok_per_w,), jnp.int32),
        pltpu.VMEM((tok_per_w,), jnp.int32),
    ]
    scratch += [pltpu.VMEM((CHUNK, D_MODEL), jnp.float32) for _ in range(NBUF)]
    scratch += [
        pltpu.VMEM((N_KINDS, D_MODEL), jnp.float32),
        pltpu.VMEM((1, D_MODEL), jnp.float32),
    ]
    scratch += [pltpu.SemaphoreType.DMA for _ in range(2 * NBUF)]

    @functools.partial(
        pl.kernel,
        mesh=mesh,
        out_type=jax.ShapeDtypeStruct((num_tokens, D_MODEL), jnp.float32),
        scratch_types=scratch,
    )
    def sc_kernel(t_hbm, kind_hbm, time_hbm, view_hbm, kind_emb_hbm, out_hbm,
                  t_all, k_all, *rest):
        bufs = rest[:NBUF]
        comb = rest[NBUF]
        viewv = rest[NBUF + 1]
        gsem = rest[NBUF + 2:2 * NBUF + 2]
        wsem = rest[2 * NBUF + 2:]
        wid = lax.axis_index("s") * nc + lax.axis_index("c")
        w_base = wid * tok_per_w

        pltpu.sync_copy(t_hbm.at[pl.ds(w_base, tok_per_w)], t_all)

        def gather(c, b):
            tv = t_all[pl.ds(c * CHUNK, CHUNK)]
            return pltpu.async_copy(time_hbm.at[tv], bufs[b], gsem[b])

        for c0 in range(PREF):
            gather(c0, c0)

        pltpu.sync_copy(kind_hbm.at[pl.ds(w_base, tok_per_w)], k_all)
        pltpu.sync_copy(kind_emb_hbm, comb)
        pltpu.sync_copy(view_hbm, viewv)
        for d in range(D_VECS):
            vv = viewv[0, pl.ds(d * LANES, LANES)]
            for k in range(N_KINDS):
                plsc.addupdate(comb.at[k, pl.ds(d * LANES, LANES)], vv)

        def emit_turn(c, u, is_static):
            bg = (u + PREF) % NBUF

            def drain():
                pltpu.make_async_copy(bufs[bg], out_hbm.at[pl.ds(0, CHUNK)],
                                      wsem[bg]).wait()

            def prefetch():
                gather(c + PREF, bg)

            if is_static:
                if c + PREF >= NBUF:
                    drain()
                if c + PREF < n_chunks:
                    prefetch()
            else:
                pl.when(c + PREF >= NBUF)(drain)
                pl.when(c + PREF < n_chunks)(prefetch)

            pltpu.make_async_copy(time_hbm.at[t_all[pl.ds(0, CHUNK)]],
                                  bufs[u], gsem[u]).wait()
            kvec = k_all[pl.ds(c * CHUNK, CHUNK)]
            kjs = [kvec[l] for l in range(LANES)]

            def d_body(d, c2, _kjs=kjs, _u=u):
                sl = pl.ds(d * LANES, LANES)
                vals = [comb[_kjs[l], sl] for l in range(LANES)]
                for l in range(LANES):
                    plsc.addupdate(bufs[_u].at[l, sl], vals[l])
                return c2

            lax.fori_loop(0, D_VECS, d_body, 0, unroll=4)
            pltpu.async_copy(bufs[u],
                             out_hbm.at[pl.ds(w_base + c * CHUNK, CHUNK)],
                             wsem[u])

        def group_body(g, carry):
            for u in range(NBUF):
                emit_turn(g * NBUF + u, u, False)
            return carry

        lax.fori_loop(0, n_groups, group_body, 0, unroll=False)
        for i in range(n_tail):
            emit_turn(n_groups * NBUF + i, i, True)

        for w in range(n_chunks - (NBUF - PREF), n_chunks):
            pltpu.make_async_copy(bufs[w % NBUF], out_hbm.at[pl.ds(0, CHUNK)],
                                  wsem[w % NBUF]).wait()

    return sc_kernel


def kernel(t, kind_id, view_id, time_emb, view_emb, kind_emb):
    b, s = t.shape
    n = b * s
    t_flat = t.reshape(n).astype(jnp.int32)
    kind_flat = kind_id.reshape(n).astype(jnp.int32)
    del view_id
    out = _make_sc_kernel(n)(t_flat, kind_flat, time_emb, view_emb, kind_emb)
    return out.reshape(b, s, D_MODEL)
